# Initial kernel scaffold; baseline (speedup 1.0000x reference)
import jax
import jax.numpy as jnp
from jax.experimental import pallas as pl


def kernel(verts_normals, faces, pix_to_face, bary_coords):
    raise NotImplementedError("write your pallas kernel here")


# speedup vs baseline: 6.7154x; 6.7154x over previous
import functools

import jax
import jax.numpy as jnp
from jax import lax
from jax.experimental import pallas as pl
from jax.experimental.pallas import tpu as pltpu
from jax.experimental.pallas import tpu_sc as plsc

NW = 32
CH1 = 1600
CH2 = 1024


def _wid():
    return lax.axis_index("s") * 2 + lax.axis_index("c")


def _stage1_body(nch, verts8, fx, fy, fz, fn, idx0, idx1, idx2,
                 r0, r1, r2, fnc, sem):
    wid = _wid()
    nmax = (nch + NW - 1) // NW
    iota = lax.iota(jnp.int32, 16)
    iota16 = iota * 16

    idxs = [idx0, idx1, idx2]
    rows = [r0, r1, r2]
    fsrcs = [fx, fy, fz]

    def body(i, carry):
        c = wid + i * NW

        @pl.when(c < nch)
        def _():
            off = c * CH1
            for j in range(3):
                pltpu.sync_copy(fsrcs[j].at[pl.ds(off, CH1)], idxs[j])
            cps = [pltpu.async_copy(verts8.at[idxs[j]], rows[j], sem)
                   for j in range(3)]
            for cp in cps:
                cp.wait()

            def asm(g, c2):
                g16 = iota + 16 * g
                d16 = iota16 + 256 * g
                for j in range(3):
                    for cc in range(3):
                        v = plsc.load_gather(rows[j], [g16, jnp.full(
                            (16,), cc, jnp.int32)])
                        plsc.store_scatter(fnc, [d16 + (4 * j + cc)], v)
                return c2

            lax.fori_loop(0, CH1 // 16, asm, 0)
            pltpu.sync_copy(fnc, fn.at[pl.ds(off * 16, CH1 * 16)])

        return carry

    lax.fori_loop(0, nmax, body, 0)


def _stage2_body(npix, fn, pix, bary, out, pidx, fnrows, bary_v, out_v, sem):
    wid = _wid()
    per_w = npix // NW
    nchunk = per_w // CH2
    base = wid * per_w

    iota = lax.iota(jnp.int32, 16)
    cn = []
    cc = []
    for k in range(3):
        m = iota + 16 * k
        q = lax.shift_right_logical(m * 21846, 16)
        cn.append(q)
        cc.append(m - 3 * q)
    col = [[cc[k] + 4 * j for j in range(3)] for k in range(3)]
    cb = [[3 * cn[k] + j for j in range(3)] for k in range(3)]

    def chunk(i, carry):
        off = base + i * CH2
        pltpu.sync_copy(pix.at[pl.ds(off, CH2)], pidx)
        gcp = pltpu.async_copy(fn.at[pidx], fnrows, sem)
        pltpu.sync_copy(bary.at[pl.ds(3 * off, 3 * CH2)], bary_v)
        gcp.wait()

        def tbody(t, c2):
            t16 = jnp.full((16,), 16 * t, jnp.int32)
            b48 = jnp.full((16,), 48 * t, jnp.int32)
            for k in range(3):
                n_vec = t16 + cn[k]
                acc = None
                for j in range(3):
                    f = plsc.load_gather(fnrows, [n_vec, col[k][j]])
                    w = plsc.load_gather(bary_v, [b48 + cb[k][j]])
                    acc = f * w if acc is None else acc + f * w
                out_v[pl.ds(48 * t + 16 * k, 16)] = acc
            return c2

        lax.fori_loop(0, CH2 // 16, tbody, 0)
        pltpu.sync_copy(out_v, out.at[pl.ds(3 * off, 3 * CH2)])
        return carry

    lax.fori_loop(0, nchunk, chunk, 0)


def kernel(verts_normals, faces, pix_to_face, bary_coords):
    B, V, _ = verts_normals.shape
    F = faces.shape[0]
    _, H, W, K = pix_to_face.shape
    P = B * H * W

    verts = verts_normals.reshape(B * V, 3)
    verts8 = jnp.pad(verts, ((0, 0), (0, 5)))
    faces32 = faces.astype(jnp.int32)
    fx, fy, fz = faces32[:, 0], faces32[:, 1], faces32[:, 2]
    pix = pix_to_face[..., 0].astype(jnp.int32).reshape(P)
    bary = bary_coords[..., 0, :].reshape(P * 3)

    mesh = plsc.VectorSubcoreMesh(core_axis_name="c", subcore_axis_name="s",
                                  num_cores=2, num_subcores=16)
    params = pltpu.CompilerParams(use_tc_tiling_on_sc=False,
                                  needs_layout_passes=False)
    nch = F // CH1

    stage1 = pl.kernel(
        functools.partial(_stage1_body, nch),
        out_type=jax.ShapeDtypeStruct((F * 16,), jnp.float32),
        mesh=mesh,
        compiler_params=params,
        scratch_types=[
            pltpu.VMEM((CH1,), jnp.int32),
            pltpu.VMEM((CH1,), jnp.int32),
            pltpu.VMEM((CH1,), jnp.int32),
            pltpu.VMEM((CH1, 8), jnp.float32),
            pltpu.VMEM((CH1, 8), jnp.float32),
            pltpu.VMEM((CH1, 8), jnp.float32),
            pltpu.VMEM((CH1 * 16,), jnp.float32),
            pltpu.SemaphoreType.DMA,
        ],
    )
    fn = stage1(verts8, fx, fy, fz).reshape(F, 16)

    stage2 = pl.kernel(
        functools.partial(_stage2_body, P),
        out_type=jax.ShapeDtypeStruct((P * 3,), jnp.float32),
        mesh=mesh,
        compiler_params=params,
        scratch_types=[
            pltpu.VMEM((CH2,), jnp.int32),
            pltpu.VMEM((CH2, 16), jnp.float32),
            pltpu.VMEM((CH2 * 3,), jnp.float32),
            pltpu.VMEM((CH2 * 3,), jnp.float32),
            pltpu.SemaphoreType.DMA,
        ],
    )
    out = stage2(fn, pix, bary)
    return out.reshape(B, H, W, 3)


# speedup vs baseline: 6.7172x; 1.0003x over previous
import functools

import jax
import jax.numpy as jnp
from jax import lax
from jax.experimental import pallas as pl
from jax.experimental.pallas import tpu as pltpu
from jax.experimental.pallas import tpu_sc as plsc

NW = 32
CH1 = 1600
CH2 = 1024


def _wid():
    return lax.axis_index("s") * 2 + lax.axis_index("c")


def _stage1_body(nch, verts8, fx, fy, fz, fn, idx0, idx1, idx2,
                 r0, r1, r2, fnc, sem):
    wid = _wid()
    nmax = (nch + NW - 1) // NW
    iota = lax.iota(jnp.int32, 16)
    iota16 = iota * 16

    idxs = [idx0, idx1, idx2]
    rows = [r0, r1, r2]
    fsrcs = [fx, fy, fz]

    def body(i, carry):
        c = wid + i * NW

        @pl.when(c < nch)
        def _():
            off = c * CH1
            for j in range(3):
                pltpu.sync_copy(fsrcs[j].at[pl.ds(off, CH1)], idxs[j])
            cps = [pltpu.async_copy(verts8.at[idxs[j]], rows[j], sem)
                   for j in range(3)]
            for cp in cps:
                cp.wait()

            def asm(g, c2):
                g16 = iota + 16 * g
                for j in range(3):
                    for cc in range(3):
                        v = plsc.load_gather(rows[j], [g16, jnp.full(
                            (16,), cc, jnp.int32)])
                        plsc.store_scatter(
                            fnc, [g16, jnp.full((16,), 4 * j + cc, jnp.int32)],
                            v)
                return c2

            lax.fori_loop(0, CH1 // 16, asm, 0)
            pltpu.sync_copy(fnc, fn.at[pl.ds(off, CH1)])

        return carry

    lax.fori_loop(0, nmax, body, 0)


def _stage2_body(npix, fn, pix, bary, out, pidx, fnrows, bary_v, out_v, sem):
    wid = _wid()
    per_w = npix // NW
    nchunk = per_w // CH2
    base = wid * per_w

    iota = lax.iota(jnp.int32, 16)
    cn = []
    cc = []
    for k in range(3):
        m = iota + 16 * k
        q = lax.shift_right_logical(m * 21846, 16)
        cn.append(q)
        cc.append(m - 3 * q)
    col = [[cc[k] + 4 * j for j in range(3)] for k in range(3)]
    cb = [[3 * cn[k] + j for j in range(3)] for k in range(3)]

    def chunk(i, carry):
        off = base + i * CH2
        pltpu.sync_copy(pix.at[pl.ds(off, CH2)], pidx)
        gcp = pltpu.async_copy(fn.at[pidx], fnrows, sem)
        pltpu.sync_copy(bary.at[pl.ds(3 * off, 3 * CH2)], bary_v)
        gcp.wait()

        def tbody(t, c2):
            t16 = jnp.full((16,), 16 * t, jnp.int32)
            b48 = jnp.full((16,), 48 * t, jnp.int32)
            for k in range(3):
                n_vec = t16 + cn[k]
                acc = None
                for j in range(3):
                    f = plsc.load_gather(fnrows, [n_vec, col[k][j]])
                    w = plsc.load_gather(bary_v, [b48 + cb[k][j]])
                    acc = f * w if acc is None else acc + f * w
                out_v[pl.ds(48 * t + 16 * k, 16)] = acc
            return c2

        lax.fori_loop(0, CH2 // 16, tbody, 0)
        pltpu.sync_copy(out_v, out.at[pl.ds(3 * off, 3 * CH2)])
        return carry

    lax.fori_loop(0, nchunk, chunk, 0)


def kernel(verts_normals, faces, pix_to_face, bary_coords):
    B, V, _ = verts_normals.shape
    F = faces.shape[0]
    _, H, W, K = pix_to_face.shape
    P = B * H * W

    verts = verts_normals.reshape(B * V, 3)
    verts8 = jnp.pad(verts, ((0, 0), (0, 5)))
    faces32 = faces.astype(jnp.int32)
    fx, fy, fz = faces32[:, 0], faces32[:, 1], faces32[:, 2]
    pix = pix_to_face[..., 0].astype(jnp.int32).reshape(P)
    bary = bary_coords[..., 0, :].reshape(P * 3)

    mesh = plsc.VectorSubcoreMesh(core_axis_name="c", subcore_axis_name="s",
                                  num_cores=2, num_subcores=16)
    params = pltpu.CompilerParams(use_tc_tiling_on_sc=False,
                                  needs_layout_passes=False)
    nch = F // CH1

    stage1 = pl.kernel(
        functools.partial(_stage1_body, nch),
        out_type=jax.ShapeDtypeStruct((F, 16), jnp.float32),
        mesh=mesh,
        compiler_params=params,
        scratch_types=[
            pltpu.VMEM((CH1,), jnp.int32),
            pltpu.VMEM((CH1,), jnp.int32),
            pltpu.VMEM((CH1,), jnp.int32),
            pltpu.VMEM((CH1, 8), jnp.float32),
            pltpu.VMEM((CH1, 8), jnp.float32),
            pltpu.VMEM((CH1, 8), jnp.float32),
            pltpu.VMEM((CH1, 16), jnp.float32),
            pltpu.SemaphoreType.DMA,
        ],
    )
    fn = stage1(verts8, fx, fy, fz)

    stage2 = pl.kernel(
        functools.partial(_stage2_body, P),
        out_type=jax.ShapeDtypeStruct((P * 3,), jnp.float32),
        mesh=mesh,
        compiler_params=params,
        scratch_types=[
            pltpu.VMEM((CH2,), jnp.int32),
            pltpu.VMEM((CH2, 16), jnp.float32),
            pltpu.VMEM((CH2 * 3,), jnp.float32),
            pltpu.VMEM((CH2 * 3,), jnp.float32),
            pltpu.SemaphoreType.DMA,
        ],
    )
    out = stage2(fn, pix, bary)
    return out.reshape(B, H, W, 3)


# speedup vs baseline: 55.3637x; 8.2421x over previous
import functools

import jax
import jax.numpy as jnp
from jax import lax
from jax.experimental import pallas as pl
from jax.experimental.pallas import tpu as pltpu
from jax.experimental.pallas import tpu_sc as plsc

NW = 32
CH1 = 1600
CH2 = 1024


def _wid():
    return lax.axis_index("s") * 2 + lax.axis_index("c")


def _stage1_body(nch, verts8, fx, fy, fz, fn, idx0, idx1, idx2,
                 r0, r1, r2, fnc, sem):
    wid = _wid()
    nmax = (nch + NW - 1) // NW
    iota = lax.iota(jnp.int32, 16)
    iota16 = iota * 16

    idxs = [idx0, idx1, idx2]
    rows = [r0, r1, r2]
    fsrcs = [fx, fy, fz]

    def body(i, carry):
        c = wid + i * NW

        @pl.when(c < nch)
        def _():
            off = c * CH1
            for j in range(3):
                pltpu.sync_copy(fsrcs[j].at[pl.ds(off, CH1)], idxs[j])
            cps = [pltpu.async_copy(verts8.at[idxs[j]], rows[j], sem)
                   for j in range(3)]
            for cp in cps:
                cp.wait()

            def asm(g, c2):
                g16 = iota + 16 * g
                for j in range(3):
                    for cc in range(3):
                        v = plsc.load_gather(rows[j], [g16, jnp.full(
                            (16,), cc, jnp.int32)])
                        plsc.store_scatter(
                            fnc, [g16, jnp.full((16,), 4 * j + cc, jnp.int32)],
                            v)
                return c2

            lax.fori_loop(0, CH1 // 16, asm, 0)
            pltpu.sync_copy(fnc, fn.at[pl.ds(off, CH1)])

        return carry

    lax.fori_loop(0, nmax, body, 0)


def _stage2_body(npix, hw, w, fn, pix, bary, out,
                 pidx, fnrows, bary_v, o0, o1, o2, sem):
    wid = _wid()
    per_w = npix // NW
    nchunk = per_w // CH2
    base = wid * per_w
    hw3 = 3 * hw
    gpr = w // 16

    iota = lax.iota(jnp.int32, 16)
    colc = [[jnp.full((16,), 4 * j + c, jnp.int32) for j in range(3)]
            for c in range(3)]
    outs = [o0, o1, o2]

    def chunk(i, carry):
        off = base + i * CH2
        b = off // hw
        r = off - b * hw
        pltpu.sync_copy(pix.at[pl.ds(off, CH2)], pidx)
        gcp = pltpu.async_copy(fn.at[pidx], fnrows, sem)
        bsrc = b * hw3 + (r // w) * (3 * w)
        pltpu.sync_copy(bary.at[pl.ds(bsrc, 3 * CH2)], bary_v)
        gcp.wait()

        def grp(g, c2):
            lr = g // gpr
            wb = 16 * g - lr * w
            n_vec = jnp.full((16,), 16 * g, jnp.int32) + iota
            bb = lr * (3 * w) + wb
            wj = [bary_v[pl.ds(bb + j * w, 16)] for j in range(3)]
            for c in range(3):
                acc = None
                for j in range(3):
                    f = plsc.load_gather(fnrows, [n_vec, colc[c][j]])
                    acc = f * wj[j] if acc is None else acc + f * wj[j]
                outs[c][pl.ds(16 * g, 16)] = acc
            return c2

        lax.fori_loop(0, CH2 // 16, grp, 0)
        for c in range(3):
            pltpu.sync_copy(outs[c], out.at[pl.ds(b * hw3 + c * hw + r, CH2)])
        return carry

    lax.fori_loop(0, nchunk, chunk, 0)


def kernel(verts_normals, faces, pix_to_face, bary_coords):
    B, V, _ = verts_normals.shape
    F = faces.shape[0]
    _, H, W, K = pix_to_face.shape
    P = B * H * W

    verts = verts_normals.reshape(B * V, 3)
    verts8 = jnp.pad(verts, ((0, 0), (0, 5)))
    faces32 = faces.astype(jnp.int32)
    fx, fy, fz = faces32[:, 0], faces32[:, 1], faces32[:, 2]
    pix = pix_to_face[..., 0].astype(jnp.int32).reshape(P)
    bary = bary_coords.transpose(0, 1, 4, 3, 2).reshape(B * H * 3 * W)

    mesh = plsc.VectorSubcoreMesh(core_axis_name="c", subcore_axis_name="s",
                                  num_cores=2, num_subcores=16)
    params = pltpu.CompilerParams(use_tc_tiling_on_sc=False,
                                  needs_layout_passes=False)
    nch = F // CH1

    stage1 = pl.kernel(
        functools.partial(_stage1_body, nch),
        out_type=jax.ShapeDtypeStruct((F, 16), jnp.float32),
        mesh=mesh,
        compiler_params=params,
        scratch_types=[
            pltpu.VMEM((CH1,), jnp.int32),
            pltpu.VMEM((CH1,), jnp.int32),
            pltpu.VMEM((CH1,), jnp.int32),
            pltpu.VMEM((CH1, 8), jnp.float32),
            pltpu.VMEM((CH1, 8), jnp.float32),
            pltpu.VMEM((CH1, 8), jnp.float32),
            pltpu.VMEM((CH1, 16), jnp.float32),
            pltpu.SemaphoreType.DMA,
        ],
    )
    fn = stage1(verts8, fx, fy, fz)

    stage2 = pl.kernel(
        functools.partial(_stage2_body, P, H * W, W),
        out_type=jax.ShapeDtypeStruct((P * 3,), jnp.float32),
        mesh=mesh,
        compiler_params=params,
        scratch_types=[
            pltpu.VMEM((CH2,), jnp.int32),
            pltpu.VMEM((CH2, 16), jnp.float32),
            pltpu.VMEM((CH2 * 3,), jnp.float32),
            pltpu.VMEM((CH2,), jnp.float32),
            pltpu.VMEM((CH2,), jnp.float32),
            pltpu.VMEM((CH2,), jnp.float32),
            pltpu.SemaphoreType.DMA,
        ],
    )
    out = stage2(fn, pix, bary)
    return out.reshape(B, 3, H, W).transpose(0, 2, 3, 1)


# speedup vs baseline: 83.0436x; 1.5000x over previous
import functools

import jax
import jax.numpy as jnp
from jax import lax
from jax.experimental import pallas as pl
from jax.experimental.pallas import tpu as pltpu
from jax.experimental.pallas import tpu_sc as plsc

NW = 32
CH1 = 1600
CH2 = 1024


def _wid():
    return lax.axis_index("s") * 2 + lax.axis_index("c")


CH0 = 2000


def _stage0_body(nv, vc, v8, vx, vy, vz, v8c):
    wid = _wid()
    nch = nv // CH0
    nmax = (nch + NW - 1) // NW
    iota = lax.iota(jnp.int32, 16)
    srcs = [vx, vy, vz]

    def body(i, carry):
        ck = wid + i * NW

        @pl.when(ck < nch)
        def _():
            base = ck * CH0
            for c in range(3):
                pltpu.sync_copy(vc.at[c, pl.ds(base, CH0)], srcs[c])

            def grp(g, c2):
                g16 = iota + 16 * g
                for c in range(3):
                    v = srcs[c][pl.ds(16 * g, 16)]
                    plsc.store_scatter(
                        v8c, [g16, jnp.full((16,), c, jnp.int32)], v)
                return c2

            lax.fori_loop(0, CH0 // 16, grp, 0)
            pltpu.sync_copy(v8c, v8.at[pl.ds(base, CH0)])

        return carry

    lax.fori_loop(0, nmax, body, 0)


def _stage1_body(nch, verts8, fx, fy, fz, fn, idx0, idx1, idx2,
                 r0, r1, r2, fnc, sem):
    wid = _wid()
    nmax = (nch + NW - 1) // NW
    iota = lax.iota(jnp.int32, 16)
    iota16 = iota * 16

    idxs = [idx0, idx1, idx2]
    rows = [r0, r1, r2]
    fsrcs = [fx, fy, fz]

    def body(i, carry):
        c = wid + i * NW

        @pl.when(c < nch)
        def _():
            off = c * CH1
            for j in range(3):
                pltpu.sync_copy(fsrcs[j].at[pl.ds(off, CH1)], idxs[j])
            cps = [pltpu.async_copy(verts8.at[idxs[j]], rows[j], sem)
                   for j in range(3)]
            for cp in cps:
                cp.wait()

            def asm(g, c2):
                g16 = iota + 16 * g
                for j in range(3):
                    for cc in range(3):
                        v = plsc.load_gather(rows[j], [g16, jnp.full(
                            (16,), cc, jnp.int32)])
                        plsc.store_scatter(
                            fnc, [g16, jnp.full((16,), 4 * j + cc, jnp.int32)],
                            v)
                return c2

            lax.fori_loop(0, CH1 // 16, asm, 0)
            pltpu.sync_copy(fnc, fn.at[pl.ds(off, CH1)])

        return carry

    lax.fori_loop(0, nmax, body, 0)


def _stage2_body(npix, hw, w, fn, pix, bary, out,
                 pidx, fnrows, bary_v, o0, o1, o2, sem):
    wid = _wid()
    per_w = npix // NW
    nchunk = per_w // CH2
    base = wid * per_w
    hw3 = 3 * hw
    gpr = w // 16

    iota = lax.iota(jnp.int32, 16)
    colc = [[jnp.full((16,), 4 * j + c, jnp.int32) for j in range(3)]
            for c in range(3)]
    outs = [o0, o1, o2]

    def chunk(i, carry):
        off = base + i * CH2
        b = off // hw
        r = off - b * hw
        pltpu.sync_copy(pix.at[pl.ds(off, CH2)], pidx)
        gcp = pltpu.async_copy(fn.at[pidx], fnrows, sem)
        bsrc = b * hw3 + (r // w) * (3 * w)
        pltpu.sync_copy(bary.at[pl.ds(bsrc, 3 * CH2)], bary_v)
        gcp.wait()

        def grp(g, c2):
            lr = g // gpr
            wb = 16 * g - lr * w
            n_vec = jnp.full((16,), 16 * g, jnp.int32) + iota
            bb = lr * (3 * w) + wb
            wj = [bary_v[pl.ds(bb + j * w, 16)] for j in range(3)]
            for c in range(3):
                acc = None
                for j in range(3):
                    f = plsc.load_gather(fnrows, [n_vec, colc[c][j]])
                    acc = f * wj[j] if acc is None else acc + f * wj[j]
                outs[c][pl.ds(16 * g, 16)] = acc
            return c2

        lax.fori_loop(0, CH2 // 16, grp, 0)
        for c in range(3):
            pltpu.sync_copy(outs[c], out.at[pl.ds(b * hw3 + c * hw + r, CH2)])
        return carry

    lax.fori_loop(0, nchunk, chunk, 0)


def kernel(verts_normals, faces, pix_to_face, bary_coords):
    B, V, _ = verts_normals.shape
    F = faces.shape[0]
    _, H, W, K = pix_to_face.shape
    P = B * H * W

    vc = verts_normals.transpose(2, 0, 1).reshape(3, B * V)
    faces32 = faces.astype(jnp.int32)
    fx, fy, fz = faces32[:, 0], faces32[:, 1], faces32[:, 2]
    pix = pix_to_face[..., 0].astype(jnp.int32).reshape(P)
    bary = bary_coords.transpose(0, 1, 4, 3, 2).reshape(B * H * 3 * W)

    mesh = plsc.VectorSubcoreMesh(core_axis_name="c", subcore_axis_name="s",
                                  num_cores=2, num_subcores=16)
    params = pltpu.CompilerParams(use_tc_tiling_on_sc=False,
                                  needs_layout_passes=False)
    nch = F // CH1

    nv = B * V
    stage0 = pl.kernel(
        functools.partial(_stage0_body, nv),
        out_type=jax.ShapeDtypeStruct((nv, 8), jnp.float32),
        mesh=mesh,
        compiler_params=params,
        scratch_types=[
            pltpu.VMEM((CH0,), jnp.float32),
            pltpu.VMEM((CH0,), jnp.float32),
            pltpu.VMEM((CH0,), jnp.float32),
            pltpu.VMEM((CH0, 8), jnp.float32),
        ],
    )
    verts8 = stage0(vc)

    stage1 = pl.kernel(
        functools.partial(_stage1_body, nch),
        out_type=jax.ShapeDtypeStruct((F, 16), jnp.float32),
        mesh=mesh,
        compiler_params=params,
        scratch_types=[
            pltpu.VMEM((CH1,), jnp.int32),
            pltpu.VMEM((CH1,), jnp.int32),
            pltpu.VMEM((CH1,), jnp.int32),
            pltpu.VMEM((CH1, 8), jnp.float32),
            pltpu.VMEM((CH1, 8), jnp.float32),
            pltpu.VMEM((CH1, 8), jnp.float32),
            pltpu.VMEM((CH1, 16), jnp.float32),
            pltpu.SemaphoreType.DMA,
        ],
    )
    fn = stage1(verts8, fx, fy, fz)

    stage2 = pl.kernel(
        functools.partial(_stage2_body, P, H * W, W),
        out_type=jax.ShapeDtypeStruct((P * 3,), jnp.float32),
        mesh=mesh,
        compiler_params=params,
        scratch_types=[
            pltpu.VMEM((CH2,), jnp.int32),
            pltpu.VMEM((CH2, 16), jnp.float32),
            pltpu.VMEM((CH2 * 3,), jnp.float32),
            pltpu.VMEM((CH2,), jnp.float32),
            pltpu.VMEM((CH2,), jnp.float32),
            pltpu.VMEM((CH2,), jnp.float32),
            pltpu.SemaphoreType.DMA,
        ],
    )
    out = stage2(fn, pix, bary)
    return out.reshape(B, 3, H, W).transpose(0, 2, 3, 1)


# speedup vs baseline: 94.2070x; 1.1344x over previous
import functools

import jax
import jax.numpy as jnp
from jax import lax
from jax.experimental import pallas as pl
from jax.experimental.pallas import tpu as pltpu
from jax.experimental.pallas import tpu_sc as plsc

NW = 32
CH1 = 1600
CH2 = 1024


def _wid():
    return lax.axis_index("s") * 2 + lax.axis_index("c")


CH0 = 2000


def _stage0_body(nv, vc, v8, vx, vy, vz, v8c):
    wid = _wid()
    nch = nv // CH0
    nmax = (nch + NW - 1) // NW
    iota = lax.iota(jnp.int32, 16)
    srcs = [vx, vy, vz]

    def body(i, carry):
        ck = wid + i * NW

        @pl.when(ck < nch)
        def _():
            base = ck * CH0
            for c in range(3):
                pltpu.sync_copy(vc.at[c, pl.ds(base, CH0)], srcs[c])

            def grp(g, c2):
                g16 = iota + 16 * g
                for c in range(3):
                    v = srcs[c][pl.ds(16 * g, 16)]
                    plsc.store_scatter(
                        v8c, [g16, jnp.full((16,), c, jnp.int32)], v)
                return c2

            lax.fori_loop(0, CH0 // 16, grp, 0)
            pltpu.sync_copy(v8c, v8.at[pl.ds(base, CH0)])

        return carry

    lax.fori_loop(0, nmax, body, 0)


def _stage1_body(nch, verts8, fx, fy, fz, fn, idx0, idx1, idx2,
                 r0, r1, r2, fnc, sem):
    wid = _wid()
    nmax = (nch + NW - 1) // NW
    iota = lax.iota(jnp.int32, 16)
    iota16 = iota * 16

    idxs = [idx0, idx1, idx2]
    rows = [r0, r1, r2]
    fsrcs = [fx, fy, fz]

    def body(i, carry):
        c = wid + i * NW

        @pl.when(c < nch)
        def _():
            off = c * CH1
            for j in range(3):
                pltpu.sync_copy(fsrcs[j].at[pl.ds(off, CH1)], idxs[j])
            cps = [pltpu.async_copy(verts8.at[idxs[j]], rows[j], sem)
                   for j in range(3)]
            for cp in cps:
                cp.wait()

            def asm(g, c2):
                g16 = iota + 16 * g
                for j in range(3):
                    for cc in range(3):
                        v = plsc.load_gather(rows[j], [g16, jnp.full(
                            (16,), cc, jnp.int32)])
                        plsc.store_scatter(
                            fnc, [g16, jnp.full((16,), 4 * j + cc, jnp.int32)],
                            v)
                return c2

            lax.fori_loop(0, CH1 // 16, asm, 0)
            pltpu.sync_copy(fnc, fn.at[pl.ds(off, CH1)])

        return carry

    lax.fori_loop(0, nmax, body, 0)


def _stage2_body(npix, hw, w, fn, pix, bary, out,
                 pidx, fnrows, bary_v, ov, gsem, ssem):
    wid = _wid()
    per_w = npix // NW
    nchunk = per_w // CH2
    base = wid * per_w
    hw3 = 3 * hw
    gpr = w // 16

    iota = lax.iota(jnp.int32, 16)
    colc = [[jnp.full((16,), 4 * j + c, jnp.int32) for j in range(3)]
            for c in range(3)]

    def issue(i, s):
        off = base + i * CH2
        b = off // hw
        r = off - b * hw
        pltpu.sync_copy(pix.at[pl.ds(off, CH2)], pidx.at[s])
        cp = pltpu.async_copy(fn.at[pidx.at[s]], fnrows.at[s], gsem.at[s])
        bsrc = b * hw3 + (r // w) * (3 * w)
        pltpu.sync_copy(bary.at[pl.ds(bsrc, 3 * CH2)], bary_v.at[s])
        return cp

    def consume(i, s):
        off = base + i * CH2
        b = off // hw
        r = off - b * hw
        pltpu.make_async_copy(fn.at[pidx.at[s]], fnrows.at[s],
                              gsem.at[s]).wait()

        def grp(g, c2):
            lr = g // gpr
            wb = 16 * g - lr * w
            n_vec = jnp.full((16,), 16 * g, jnp.int32) + iota
            bb = lr * (3 * w) + wb
            wj = [bary_v.at[s][pl.ds(bb + j * w, 16)] for j in range(3)]
            for c in range(3):
                acc = None
                for j in range(3):
                    f = plsc.load_gather(fnrows.at[s], [n_vec, colc[c][j]])
                    acc = f * wj[j] if acc is None else acc + f * wj[j]
                ov.at[s, c][pl.ds(16 * g, 16)] = acc
            return c2

        lax.fori_loop(0, CH2 // 16, grp, 0)
        for c in range(3):
            pltpu.async_copy(ov.at[s, c],
                             out.at[pl.ds(b * hw3 + c * hw + r, CH2)],
                             ssem.at[s])

    def drain(s):
        for c in range(3):
            pltpu.make_async_copy(ov.at[s, c], out.at[pl.ds(0, CH2)],
                                  ssem.at[s]).wait()

    issue(0, 0)

    def pair(i2, carry):
        i = 2 * i2
        issue(i + 1, 1)

        @pl.when(i2 > 0)
        def _():
            drain(0)
        consume(i, 0)

        @pl.when(i + 2 < nchunk)
        def _():
            issue(i + 2, 0)

        @pl.when(i2 > 0)
        def _():
            drain(1)
        consume(i + 1, 1)
        return carry

    lax.fori_loop(0, nchunk // 2, pair, 0)
    drain(0)
    drain(1)


def kernel(verts_normals, faces, pix_to_face, bary_coords):
    B, V, _ = verts_normals.shape
    F = faces.shape[0]
    _, H, W, K = pix_to_face.shape
    P = B * H * W

    vc = verts_normals.transpose(2, 0, 1).reshape(3, B * V)
    faces32 = faces.astype(jnp.int32)
    fx, fy, fz = faces32[:, 0], faces32[:, 1], faces32[:, 2]
    pix = pix_to_face[..., 0].astype(jnp.int32).reshape(P)
    bary = bary_coords.transpose(0, 1, 4, 3, 2).reshape(B * H * 3 * W)

    mesh = plsc.VectorSubcoreMesh(core_axis_name="c", subcore_axis_name="s",
                                  num_cores=2, num_subcores=16)
    params = pltpu.CompilerParams(use_tc_tiling_on_sc=False,
                                  needs_layout_passes=False)
    nch = F // CH1

    nv = B * V
    stage0 = pl.kernel(
        functools.partial(_stage0_body, nv),
        out_type=jax.ShapeDtypeStruct((nv, 8), jnp.float32),
        mesh=mesh,
        compiler_params=params,
        scratch_types=[
            pltpu.VMEM((CH0,), jnp.float32),
            pltpu.VMEM((CH0,), jnp.float32),
            pltpu.VMEM((CH0,), jnp.float32),
            pltpu.VMEM((CH0, 8), jnp.float32),
        ],
    )
    verts8 = stage0(vc)

    stage1 = pl.kernel(
        functools.partial(_stage1_body, nch),
        out_type=jax.ShapeDtypeStruct((F, 16), jnp.float32),
        mesh=mesh,
        compiler_params=params,
        scratch_types=[
            pltpu.VMEM((CH1,), jnp.int32),
            pltpu.VMEM((CH1,), jnp.int32),
            pltpu.VMEM((CH1,), jnp.int32),
            pltpu.VMEM((CH1, 8), jnp.float32),
            pltpu.VMEM((CH1, 8), jnp.float32),
            pltpu.VMEM((CH1, 8), jnp.float32),
            pltpu.VMEM((CH1, 16), jnp.float32),
            pltpu.SemaphoreType.DMA,
        ],
    )
    fn = stage1(verts8, fx, fy, fz)

    stage2 = pl.kernel(
        functools.partial(_stage2_body, P, H * W, W),
        out_type=jax.ShapeDtypeStruct((P * 3,), jnp.float32),
        mesh=mesh,
        compiler_params=params,
        scratch_types=[
            pltpu.VMEM((2, CH2), jnp.int32),
            pltpu.VMEM((2, CH2, 16), jnp.float32),
            pltpu.VMEM((2, CH2 * 3), jnp.float32),
            pltpu.VMEM((2, 3, CH2), jnp.float32),
            pltpu.SemaphoreType.DMA((2,)),
            pltpu.SemaphoreType.DMA((2,)),
        ],
    )
    out = stage2(fn, pix, bary)
    return out.reshape(B, 3, H, W).transpose(0, 2, 3, 1)


# speedup vs baseline: 106.2518x; 1.1279x over previous
import functools

import jax
import jax.numpy as jnp
from jax import lax
from jax.experimental import pallas as pl
from jax.experimental.pallas import tpu as pltpu
from jax.experimental.pallas import tpu_sc as plsc

NW = 32
CH1 = 800
CH2 = 1024


def _wid():
    return lax.axis_index("s") * 2 + lax.axis_index("c")


CH0 = 2000


def _stage0_body(nv, vc, v8, vx, vy, vz, v8c):
    wid = _wid()
    nch = nv // CH0
    nmax = (nch + NW - 1) // NW
    iota = lax.iota(jnp.int32, 16)
    srcs = [vx, vy, vz]

    def body(i, carry):
        ck = wid + i * NW

        @pl.when(ck < nch)
        def _():
            base = ck * CH0
            for c in range(3):
                pltpu.sync_copy(vc.at[c, pl.ds(base, CH0)], srcs[c])

            def grp(g, c2):
                g16 = iota + 16 * g
                for c in range(3):
                    v = srcs[c][pl.ds(16 * g, 16)]
                    plsc.store_scatter(
                        v8c, [g16, jnp.full((16,), c, jnp.int32)], v)
                return c2

            lax.fori_loop(0, CH0 // 16, grp, 0)
            pltpu.sync_copy(v8c, v8.at[pl.ds(base, CH0)])

        return carry

    lax.fori_loop(0, nmax, body, 0)


def _stage1_body(nch, verts8, fx, fy, fz, fn, idx3, rows, fnc, gsem, ssem):
    wid = _wid()
    nmax = (nch + NW - 1) // NW
    if nmax % 2:
        nmax += 1
    iota = lax.iota(jnp.int32, 16)
    fsrcs = [fx, fy, fz]

    def issue(i, s):
        c = wid + i * NW

        @pl.when(c < nch)
        def _():
            off = c * CH1
            for j in range(3):
                pltpu.sync_copy(fsrcs[j].at[pl.ds(off, CH1)], idx3.at[s, j])
            for j in range(3):
                pltpu.async_copy(verts8.at[idx3.at[s, j]], rows.at[s, j],
                                 gsem.at[s])

    def consume(i, s):
        c = wid + i * NW

        @pl.when(c < nch)
        def _():
            off = c * CH1
            for j in range(3):
                pltpu.make_async_copy(verts8.at[idx3.at[s, j]],
                                      rows.at[s, j], gsem.at[s]).wait()

            def asm(g, c2):
                g16 = iota + 16 * g
                for j in range(3):
                    for cc in range(3):
                        v = plsc.load_gather(rows.at[s, j], [g16, jnp.full(
                            (16,), cc, jnp.int32)])
                        plsc.store_scatter(
                            fnc.at[s],
                            [g16, jnp.full((16,), 4 * j + cc, jnp.int32)], v)
                return c2

            lax.fori_loop(0, CH1 // 16, asm, 0)
            pltpu.async_copy(fnc.at[s], fn.at[pl.ds(off, CH1)], ssem.at[s])

    def drain(i, s):
        c = wid + i * NW

        @pl.when(jnp.logical_and(i >= 0, c < nch))
        def _():
            pltpu.make_async_copy(fnc.at[s], fn.at[pl.ds(0, CH1)],
                                  ssem.at[s]).wait()

    issue(0, 0)

    def pair(i2, carry):
        i = 2 * i2
        issue(i + 1, 1)
        drain(i - 2, 0)
        consume(i, 0)
        issue(i + 2, 0)
        drain(i - 1, 1)
        consume(i + 1, 1)
        return carry

    lax.fori_loop(0, nmax // 2, pair, 0)
    drain(nmax - 2, 0)
    drain(nmax - 1, 1)


def _stage2_body(npix, hw, w, fn, pix, bary, out,
                 pidx, fnrows, bary_v, ov, gsem, ssem):
    wid = _wid()
    per_w = npix // NW
    nchunk = per_w // CH2
    base = wid * per_w
    hw3 = 3 * hw
    gpr = w // 16

    iota = lax.iota(jnp.int32, 16)
    colc = [[jnp.full((16,), 4 * j + c, jnp.int32) for j in range(3)]
            for c in range(3)]

    def issue(i, s):
        off = base + i * CH2
        b = off // hw
        r = off - b * hw
        pltpu.sync_copy(pix.at[pl.ds(off, CH2)], pidx.at[s])
        cp = pltpu.async_copy(fn.at[pidx.at[s]], fnrows.at[s], gsem.at[s])
        bsrc = b * hw3 + (r // w) * (3 * w)
        pltpu.sync_copy(bary.at[pl.ds(bsrc, 3 * CH2)], bary_v.at[s])
        return cp

    def consume(i, s):
        off = base + i * CH2
        b = off // hw
        r = off - b * hw
        pltpu.make_async_copy(fn.at[pidx.at[s]], fnrows.at[s],
                              gsem.at[s]).wait()

        def grp(g, c2):
            lr = g // gpr
            wb = 16 * g - lr * w
            n_vec = jnp.full((16,), 16 * g, jnp.int32) + iota
            bb = lr * (3 * w) + wb
            wj = [bary_v.at[s][pl.ds(bb + j * w, 16)] for j in range(3)]
            for c in range(3):
                acc = None
                for j in range(3):
                    f = plsc.load_gather(fnrows.at[s], [n_vec, colc[c][j]])
                    acc = f * wj[j] if acc is None else acc + f * wj[j]
                ov.at[s, c][pl.ds(16 * g, 16)] = acc
            return c2

        lax.fori_loop(0, CH2 // 16, grp, 0)
        for c in range(3):
            pltpu.async_copy(ov.at[s, c],
                             out.at[pl.ds(b * hw3 + c * hw + r, CH2)],
                             ssem.at[s])

    def drain(s):
        for c in range(3):
            pltpu.make_async_copy(ov.at[s, c], out.at[pl.ds(0, CH2)],
                                  ssem.at[s]).wait()

    issue(0, 0)

    def pair(i2, carry):
        i = 2 * i2
        issue(i + 1, 1)

        @pl.when(i2 > 0)
        def _():
            drain(0)
        consume(i, 0)

        @pl.when(i + 2 < nchunk)
        def _():
            issue(i + 2, 0)

        @pl.when(i2 > 0)
        def _():
            drain(1)
        consume(i + 1, 1)
        return carry

    lax.fori_loop(0, nchunk // 2, pair, 0)
    drain(0)
    drain(1)


def kernel(verts_normals, faces, pix_to_face, bary_coords):
    B, V, _ = verts_normals.shape
    F = faces.shape[0]
    _, H, W, K = pix_to_face.shape
    P = B * H * W

    vc = verts_normals.transpose(2, 0, 1).reshape(3, B * V)
    faces32 = faces.astype(jnp.int32)
    fx, fy, fz = faces32[:, 0], faces32[:, 1], faces32[:, 2]
    pix = pix_to_face[..., 0].astype(jnp.int32).reshape(P)
    bary = bary_coords.transpose(0, 1, 4, 3, 2).reshape(B * H * 3 * W)

    mesh = plsc.VectorSubcoreMesh(core_axis_name="c", subcore_axis_name="s",
                                  num_cores=2, num_subcores=16)
    params = pltpu.CompilerParams(use_tc_tiling_on_sc=False,
                                  needs_layout_passes=False)
    nch = F // CH1

    nv = B * V
    stage0 = pl.kernel(
        functools.partial(_stage0_body, nv),
        out_type=jax.ShapeDtypeStruct((nv, 8), jnp.float32),
        mesh=mesh,
        compiler_params=params,
        scratch_types=[
            pltpu.VMEM((CH0,), jnp.float32),
            pltpu.VMEM((CH0,), jnp.float32),
            pltpu.VMEM((CH0,), jnp.float32),
            pltpu.VMEM((CH0, 8), jnp.float32),
        ],
    )
    verts8 = stage0(vc)

    stage1 = pl.kernel(
        functools.partial(_stage1_body, nch),
        out_type=jax.ShapeDtypeStruct((F, 16), jnp.float32),
        mesh=mesh,
        compiler_params=params,
        scratch_types=[
            pltpu.VMEM((2, 3, CH1), jnp.int32),
            pltpu.VMEM((2, 3, CH1, 8), jnp.float32),
            pltpu.VMEM((2, CH1, 16), jnp.float32),
            pltpu.SemaphoreType.DMA((2,)),
            pltpu.SemaphoreType.DMA((2,)),
        ],
    )
    fn = stage1(verts8, fx, fy, fz)

    stage2 = pl.kernel(
        functools.partial(_stage2_body, P, H * W, W),
        out_type=jax.ShapeDtypeStruct((P * 3,), jnp.float32),
        mesh=mesh,
        compiler_params=params,
        scratch_types=[
            pltpu.VMEM((2, CH2), jnp.int32),
            pltpu.VMEM((2, CH2, 16), jnp.float32),
            pltpu.VMEM((2, CH2 * 3), jnp.float32),
            pltpu.VMEM((2, 3, CH2), jnp.float32),
            pltpu.SemaphoreType.DMA((2,)),
            pltpu.SemaphoreType.DMA((2,)),
        ],
    )
    out = stage2(fn, pix, bary)
    return out.reshape(B, 3, H, W).transpose(0, 2, 3, 1)


# speedup vs baseline: 115.6556x; 1.0885x over previous
import functools

import jax
import jax.numpy as jnp
from jax import lax
from jax.experimental import pallas as pl
from jax.experimental.pallas import tpu as pltpu
from jax.experimental.pallas import tpu_sc as plsc

NW = 32
CH1 = 800
CH2 = 1024


def _wid():
    return lax.axis_index("s") * 2 + lax.axis_index("c")


CH0 = 2000


def _stage0_body(nv, vc, v8, vx, vy, vz, v8c):
    wid = _wid()
    nch = nv // CH0
    nmax = (nch + NW - 1) // NW
    iota = lax.iota(jnp.int32, 16)
    srcs = [vx, vy, vz]

    def body(i, carry):
        ck = wid + i * NW

        @pl.when(ck < nch)
        def _():
            base = ck * CH0
            for c in range(3):
                pltpu.sync_copy(vc.at[c, pl.ds(base, CH0)], srcs[c])

            def grp(g, c2):
                g16 = iota + 16 * g
                for c in range(3):
                    v = srcs[c][pl.ds(16 * g, 16)]
                    plsc.store_scatter(
                        v8c, [g16, jnp.full((16,), c, jnp.int32)], v)
                return c2

            lax.fori_loop(0, CH0 // 16, grp, 0)
            pltpu.sync_copy(v8c, v8.at[pl.ds(base, CH0)])

        return carry

    lax.fori_loop(0, nmax, body, 0)


def _stage1_body(nch, verts8, fx, fy, fz, fn, idx3, rows, fnc, gsem, ssem):
    wid = _wid()
    nmax = (nch + NW - 1) // NW
    if nmax % 2:
        nmax += 1
    iota = lax.iota(jnp.int32, 16)
    fsrcs = [fx, fy, fz]

    def issue(i, s):
        c = wid + i * NW

        @pl.when(c < nch)
        def _():
            off = c * CH1
            for j in range(3):
                pltpu.sync_copy(fsrcs[j].at[pl.ds(off, CH1)], idx3.at[s, j])
            for j in range(3):
                pltpu.async_copy(verts8.at[idx3.at[s, j]], rows.at[s, j],
                                 gsem.at[s])

    def consume(i, s):
        c = wid + i * NW

        @pl.when(c < nch)
        def _():
            off = c * CH1
            for j in range(3):
                pltpu.make_async_copy(verts8.at[idx3.at[s, j]],
                                      rows.at[s, j], gsem.at[s]).wait()

            def asm(g, c2):
                g16 = iota + 16 * g
                for j in range(3):
                    for cc in range(3):
                        v = plsc.load_gather(rows.at[s, j], [g16, jnp.full(
                            (16,), cc, jnp.int32)])
                        plsc.store_scatter(
                            fnc.at[s],
                            [g16, jnp.full((16,), 4 * j + cc, jnp.int32)], v)
                return c2

            lax.fori_loop(0, CH1 // 16, asm, 0)
            pltpu.async_copy(fnc.at[s], fn.at[pl.ds(off, CH1)], ssem.at[s])

    def drain(i, s):
        c = wid + i * NW

        @pl.when(jnp.logical_and(i >= 0, c < nch))
        def _():
            pltpu.make_async_copy(fnc.at[s], fn.at[pl.ds(0, CH1)],
                                  ssem.at[s]).wait()

    issue(0, 0)

    def pair(i2, carry):
        i = 2 * i2
        issue(i + 1, 1)
        drain(i - 2, 0)
        consume(i, 0)
        issue(i + 2, 0)
        drain(i - 1, 1)
        consume(i + 1, 1)
        return carry

    lax.fori_loop(0, nmax // 2, pair, 0)
    drain(nmax - 2, 0)
    drain(nmax - 1, 1)


def _stage2_body(npix, hw, w, fn, pix, bary, out,
                 pidx, fnrows, bary_v, ov, gsem, psem, bsem, ssem):
    wid = _wid()
    per_w = npix // NW
    nchunk = per_w // CH2
    base = wid * per_w
    hw3 = 3 * hw
    gpr = w // 16

    iota = lax.iota(jnp.int32, 16)
    colc = [[jnp.full((16,), 4 * j + c, jnp.int32) for j in range(3)]
            for c in range(3)]

    def load_pix(i, s):
        off = base + i * CH2
        pltpu.async_copy(pix.at[pl.ds(off, CH2)], pidx.at[s], psem.at[s])

    def wait_pix(i, s):
        off = base + i * CH2
        pltpu.make_async_copy(pix.at[pl.ds(off, CH2)], pidx.at[s],
                              psem.at[s]).wait()

    def load_bary(i, s):
        off = base + i * CH2
        b = off // hw
        r = off - b * hw
        bsrc = b * hw3 + (r // w) * (3 * w)
        pltpu.async_copy(bary.at[pl.ds(bsrc, 3 * CH2)], bary_v.at[s],
                         bsem.at[s])

    def wait_bary(s):
        pltpu.make_async_copy(bary.at[pl.ds(0, 3 * CH2)], bary_v.at[s],
                              bsem.at[s]).wait()

    def issue_gather(s):
        pltpu.async_copy(fn.at[pidx.at[s]], fnrows.at[s], gsem.at[s])

    def wait_gather(s):
        pltpu.make_async_copy(fn.at[pidx.at[s]], fnrows.at[s],
                              gsem.at[s]).wait()

    def drain_ov(s):
        for c in range(3):
            pltpu.make_async_copy(ov.at[s, c], out.at[pl.ds(0, CH2)],
                                  ssem.at[s]).wait()

    def compute(i, s):
        off = base + i * CH2
        b = off // hw
        r = off - b * hw

        def grp(g, c2):
            lr = g // gpr
            wb = 16 * g - lr * w
            n_vec = jnp.full((16,), 16 * g, jnp.int32) + iota
            bb = lr * (3 * w) + wb
            wj = [bary_v.at[s][pl.ds(bb + j * w, 16)] for j in range(3)]
            for c in range(3):
                acc = None
                for j in range(3):
                    f = plsc.load_gather(fnrows.at[s], [n_vec, colc[c][j]])
                    acc = f * wj[j] if acc is None else acc + f * wj[j]
                ov.at[s, c][pl.ds(16 * g, 16)] = acc
            return c2

        lax.fori_loop(0, CH2 // 16, grp, 0)
        for c in range(3):
            pltpu.async_copy(ov.at[s, c],
                             out.at[pl.ds(b * hw3 + c * hw + r, CH2)],
                             ssem.at[s])

    def slot(i, s, s1):
        @pl.when(i + 1 < nchunk)
        def _():
            wait_pix(i + 1, s1)
            issue_gather(s1)
        wait_gather(s)
        wait_bary(s)

        @pl.when(i >= 2)
        def _():
            drain_ov(s)
        compute(i, s)

        @pl.when(i + 2 < nchunk)
        def _():
            load_pix(i + 2, s)
            load_bary(i + 2, s)

    load_pix(0, 0)
    load_bary(0, 0)
    load_pix(1, 1)
    load_bary(1, 1)
    wait_pix(0, 0)
    issue_gather(0)

    def pair(i2, carry):
        i = 2 * i2
        slot(i, 0, 1)
        slot(i + 1, 1, 0)
        return carry

    lax.fori_loop(0, nchunk // 2, pair, 0)
    drain_ov(0)
    drain_ov(1)


def kernel(verts_normals, faces, pix_to_face, bary_coords):
    B, V, _ = verts_normals.shape
    F = faces.shape[0]
    _, H, W, K = pix_to_face.shape
    P = B * H * W

    vc = verts_normals.transpose(2, 0, 1).reshape(3, B * V)
    faces32 = faces.astype(jnp.int32)
    fx, fy, fz = faces32[:, 0], faces32[:, 1], faces32[:, 2]
    pix = pix_to_face[..., 0].astype(jnp.int32).reshape(P)
    bary = bary_coords.transpose(0, 1, 4, 3, 2).reshape(B * H * 3 * W)

    mesh = plsc.VectorSubcoreMesh(core_axis_name="c", subcore_axis_name="s",
                                  num_cores=2, num_subcores=16)
    params = pltpu.CompilerParams(use_tc_tiling_on_sc=False,
                                  needs_layout_passes=False)
    nch = F // CH1

    nv = B * V
    stage0 = pl.kernel(
        functools.partial(_stage0_body, nv),
        out_type=jax.ShapeDtypeStruct((nv, 8), jnp.float32),
        mesh=mesh,
        compiler_params=params,
        scratch_types=[
            pltpu.VMEM((CH0,), jnp.float32),
            pltpu.VMEM((CH0,), jnp.float32),
            pltpu.VMEM((CH0,), jnp.float32),
            pltpu.VMEM((CH0, 8), jnp.float32),
        ],
    )
    verts8 = stage0(vc)

    stage1 = pl.kernel(
        functools.partial(_stage1_body, nch),
        out_type=jax.ShapeDtypeStruct((F, 16), jnp.float32),
        mesh=mesh,
        compiler_params=params,
        scratch_types=[
            pltpu.VMEM((2, 3, CH1), jnp.int32),
            pltpu.VMEM((2, 3, CH1, 8), jnp.float32),
            pltpu.VMEM((2, CH1, 16), jnp.float32),
            pltpu.SemaphoreType.DMA((2,)),
            pltpu.SemaphoreType.DMA((2,)),
        ],
    )
    fn = stage1(verts8, fx, fy, fz)

    stage2 = pl.kernel(
        functools.partial(_stage2_body, P, H * W, W),
        out_type=jax.ShapeDtypeStruct((P * 3,), jnp.float32),
        mesh=mesh,
        compiler_params=params,
        scratch_types=[
            pltpu.VMEM((2, CH2), jnp.int32),
            pltpu.VMEM((2, CH2, 16), jnp.float32),
            pltpu.VMEM((2, CH2 * 3), jnp.float32),
            pltpu.VMEM((2, 3, CH2), jnp.float32),
            pltpu.SemaphoreType.DMA((2,)),
            pltpu.SemaphoreType.DMA((2,)),
            pltpu.SemaphoreType.DMA((2,)),
            pltpu.SemaphoreType.DMA((2,)),
        ],
    )
    out = stage2(fn, pix, bary)
    return out.reshape(B, 3, H, W).transpose(0, 2, 3, 1)


# speedup vs baseline: 121.0632x; 1.0468x over previous
import functools

import jax
import jax.numpy as jnp
from jax import lax
from jax.experimental import pallas as pl
from jax.experimental.pallas import tpu as pltpu
from jax.experimental.pallas import tpu_sc as plsc

NW = 32
CH1 = 800
CH2 = 1024


def _wid():
    return lax.axis_index("s") * 2 + lax.axis_index("c")


CH0 = 2000


def _stage0_body(nv, vc, v8, vx, vy, vz, v8c):
    wid = _wid()
    nch = nv // CH0
    nmax = (nch + NW - 1) // NW
    iota = lax.iota(jnp.int32, 16)
    srcs = [vx, vy, vz]

    def body(i, carry):
        ck = wid + i * NW

        @pl.when(ck < nch)
        def _():
            base = ck * CH0
            for c in range(3):
                pltpu.sync_copy(vc.at[c, pl.ds(base, CH0)], srcs[c])

            def grp(g, c2):
                g16 = iota + 16 * g
                for c in range(3):
                    v = srcs[c][pl.ds(16 * g, 16)]
                    plsc.store_scatter(
                        v8c, [g16, jnp.full((16,), c, jnp.int32)], v)
                return c2

            lax.fori_loop(0, CH0 // 16, grp, 0)
            pltpu.sync_copy(v8c, v8.at[pl.ds(base, CH0)])

        return carry

    lax.fori_loop(0, nmax, body, 0)


def _stage1_body(nch, verts8, fx, fy, fz, fn, idx3, rows, fnc,
                 isem, gsem, ssem):
    wid = _wid()
    nmax = (nch + NW - 1) // NW
    if nmax % 2:
        nmax += 1
    iota = lax.iota(jnp.int32, 16)
    fsrcs = [fx, fy, fz]

    def valid(i):
        return (wid + i * NW) < nch

    def load_idx(i, s):
        @pl.when(valid(i))
        def _():
            off = (wid + i * NW) * CH1
            for j in range(3):
                pltpu.async_copy(fsrcs[j].at[pl.ds(off, CH1)], idx3.at[s, j],
                                 isem.at[s])

    def start_gather(i, s):
        @pl.when(valid(i))
        def _():
            for j in range(3):
                pltpu.make_async_copy(fsrcs[j].at[pl.ds(0, CH1)],
                                      idx3.at[s, j], isem.at[s]).wait()
            for j in range(3):
                pltpu.async_copy(verts8.at[idx3.at[s, j]], rows.at[s, j],
                                 gsem.at[s])

    def consume(i, s):
        @pl.when(valid(i))
        def _():
            off = (wid + i * NW) * CH1
            for j in range(3):
                pltpu.make_async_copy(verts8.at[idx3.at[s, j]],
                                      rows.at[s, j], gsem.at[s]).wait()

            def asm(g, c2):
                g16 = iota + 16 * g
                for j in range(3):
                    for cc in range(3):
                        v = plsc.load_gather(rows.at[s, j], [g16, jnp.full(
                            (16,), cc, jnp.int32)])
                        plsc.store_scatter(
                            fnc.at[s],
                            [g16, jnp.full((16,), 4 * j + cc, jnp.int32)], v)
                return c2

            lax.fori_loop(0, CH1 // 16, asm, 0)
            pltpu.async_copy(fnc.at[s], fn.at[pl.ds(off, CH1)], ssem.at[s])

    def drain(i, s):
        @pl.when(jnp.logical_and(i >= 0, valid(i)))
        def _():
            pltpu.make_async_copy(fnc.at[s], fn.at[pl.ds(0, CH1)],
                                  ssem.at[s]).wait()

    load_idx(0, 0)
    load_idx(1, 1)
    start_gather(0, 0)

    def slot(i, s, s1):
        start_gather(i + 1, s1)
        drain(i - 2, s)
        consume(i, s)
        load_idx(i + 2, s)

    def pair(i2, carry):
        i = 2 * i2
        slot(i, 0, 1)
        slot(i + 1, 1, 0)
        return carry

    lax.fori_loop(0, nmax // 2, pair, 0)
    drain(nmax - 2, 0)
    drain(nmax - 1, 1)


def _stage2_body(npix, hw, w, fn, pix, bary, out,
                 pidx, fnrows, bary_v, ov, gsem, psem, bsem, ssem):
    wid = _wid()
    per_w = npix // NW
    nchunk = per_w // CH2
    base = wid * per_w
    hw3 = 3 * hw
    gpr = w // 16

    iota = lax.iota(jnp.int32, 16)
    colc = [[jnp.full((16,), 4 * j + c, jnp.int32) for j in range(3)]
            for c in range(3)]

    def load_pix(i, s):
        off = base + i * CH2
        pltpu.async_copy(pix.at[pl.ds(off, CH2)], pidx.at[s], psem.at[s])

    def wait_pix(i, s):
        off = base + i * CH2
        pltpu.make_async_copy(pix.at[pl.ds(off, CH2)], pidx.at[s],
                              psem.at[s]).wait()

    def load_bary(i, s):
        off = base + i * CH2
        b = off // hw
        r = off - b * hw
        bsrc = b * hw3 + (r // w) * (3 * w)
        pltpu.async_copy(bary.at[pl.ds(bsrc, 3 * CH2)], bary_v.at[s],
                         bsem.at[s])

    def wait_bary(s):
        pltpu.make_async_copy(bary.at[pl.ds(0, 3 * CH2)], bary_v.at[s],
                              bsem.at[s]).wait()

    def issue_gather(s):
        pltpu.async_copy(fn.at[pidx.at[s]], fnrows.at[s], gsem.at[s])

    def wait_gather(s):
        pltpu.make_async_copy(fn.at[pidx.at[s]], fnrows.at[s],
                              gsem.at[s]).wait()

    def drain_ov(s):
        for c in range(3):
            pltpu.make_async_copy(ov.at[s, c], out.at[pl.ds(0, CH2)],
                                  ssem.at[s]).wait()

    def compute(i, s):
        off = base + i * CH2
        b = off // hw
        r = off - b * hw

        def grp(g, c2):
            lr = g // gpr
            wb = 16 * g - lr * w
            n_vec = jnp.full((16,), 16 * g, jnp.int32) + iota
            bb = lr * (3 * w) + wb
            wj = [bary_v.at[s][pl.ds(bb + j * w, 16)] for j in range(3)]
            for c in range(3):
                acc = None
                for j in range(3):
                    f = plsc.load_gather(fnrows.at[s], [n_vec, colc[c][j]])
                    acc = f * wj[j] if acc is None else acc + f * wj[j]
                ov.at[s, c][pl.ds(16 * g, 16)] = acc
            return c2

        lax.fori_loop(0, CH2 // 16, grp, 0)
        for c in range(3):
            pltpu.async_copy(ov.at[s, c],
                             out.at[pl.ds(b * hw3 + c * hw + r, CH2)],
                             ssem.at[s])

    def slot(i, s, s1):
        @pl.when(i + 1 < nchunk)
        def _():
            wait_pix(i + 1, s1)
            issue_gather(s1)
        wait_gather(s)
        wait_bary(s)

        @pl.when(i >= 2)
        def _():
            drain_ov(s)
        compute(i, s)

        @pl.when(i + 2 < nchunk)
        def _():
            load_pix(i + 2, s)
            load_bary(i + 2, s)

    load_pix(0, 0)
    load_bary(0, 0)
    load_pix(1, 1)
    load_bary(1, 1)
    wait_pix(0, 0)
    issue_gather(0)

    def pair(i2, carry):
        i = 2 * i2
        slot(i, 0, 1)
        slot(i + 1, 1, 0)
        return carry

    lax.fori_loop(0, nchunk // 2, pair, 0)
    drain_ov(0)
    drain_ov(1)


def kernel(verts_normals, faces, pix_to_face, bary_coords):
    B, V, _ = verts_normals.shape
    F = faces.shape[0]
    _, H, W, K = pix_to_face.shape
    P = B * H * W

    vc = verts_normals.transpose(2, 0, 1).reshape(3, B * V)
    faces32 = faces.astype(jnp.int32)
    fx, fy, fz = faces32[:, 0], faces32[:, 1], faces32[:, 2]
    pix = pix_to_face[..., 0].astype(jnp.int32).reshape(P)
    bary = bary_coords.transpose(0, 1, 4, 3, 2).reshape(B * H * 3 * W)

    mesh = plsc.VectorSubcoreMesh(core_axis_name="c", subcore_axis_name="s",
                                  num_cores=2, num_subcores=16)
    params = pltpu.CompilerParams(use_tc_tiling_on_sc=False,
                                  needs_layout_passes=False)
    nch = F // CH1

    nv = B * V
    stage0 = pl.kernel(
        functools.partial(_stage0_body, nv),
        out_type=jax.ShapeDtypeStruct((nv, 8), jnp.float32),
        mesh=mesh,
        compiler_params=params,
        scratch_types=[
            pltpu.VMEM((CH0,), jnp.float32),
            pltpu.VMEM((CH0,), jnp.float32),
            pltpu.VMEM((CH0,), jnp.float32),
            pltpu.VMEM((CH0, 8), jnp.float32),
        ],
    )
    verts8 = stage0(vc)

    stage1 = pl.kernel(
        functools.partial(_stage1_body, nch),
        out_type=jax.ShapeDtypeStruct((F, 16), jnp.float32),
        mesh=mesh,
        compiler_params=params,
        scratch_types=[
            pltpu.VMEM((2, 3, CH1), jnp.int32),
            pltpu.VMEM((2, 3, CH1, 8), jnp.float32),
            pltpu.VMEM((2, CH1, 16), jnp.float32),
            pltpu.SemaphoreType.DMA((2,)),
            pltpu.SemaphoreType.DMA((2,)),
            pltpu.SemaphoreType.DMA((2,)),
        ],
    )
    fn = stage1(verts8, fx, fy, fz)

    stage2 = pl.kernel(
        functools.partial(_stage2_body, P, H * W, W),
        out_type=jax.ShapeDtypeStruct((P * 3,), jnp.float32),
        mesh=mesh,
        compiler_params=params,
        scratch_types=[
            pltpu.VMEM((2, CH2), jnp.int32),
            pltpu.VMEM((2, CH2, 16), jnp.float32),
            pltpu.VMEM((2, CH2 * 3), jnp.float32),
            pltpu.VMEM((2, 3, CH2), jnp.float32),
            pltpu.SemaphoreType.DMA((2,)),
            pltpu.SemaphoreType.DMA((2,)),
            pltpu.SemaphoreType.DMA((2,)),
            pltpu.SemaphoreType.DMA((2,)),
        ],
    )
    out = stage2(fn, pix, bary)
    return out.reshape(B, 3, H, W).transpose(0, 2, 3, 1)


# speedup vs baseline: 124.3537x; 1.0272x over previous
import functools

import jax
import jax.numpy as jnp
from jax import lax
from jax.experimental import pallas as pl
from jax.experimental.pallas import tpu as pltpu
from jax.experimental.pallas import tpu_sc as plsc

NW = 32
CH1 = 800
CH2 = 2048


def _wid():
    return lax.axis_index("s") * 2 + lax.axis_index("c")


CH0 = 2000


def _stage0_body(nv, vc, v8, vx, vy, vz, v8c):
    wid = _wid()
    nch = nv // CH0
    nmax = (nch + NW - 1) // NW
    iota = lax.iota(jnp.int32, 16)
    srcs = [vx, vy, vz]

    def body(i, carry):
        ck = wid + i * NW

        @pl.when(ck < nch)
        def _():
            base = ck * CH0
            for c in range(3):
                pltpu.sync_copy(vc.at[c, pl.ds(base, CH0)], srcs[c])

            def grp(g, c2):
                g16 = iota + 16 * g
                for c in range(3):
                    v = srcs[c][pl.ds(16 * g, 16)]
                    plsc.store_scatter(
                        v8c, [g16, jnp.full((16,), c, jnp.int32)], v)
                return c2

            lax.fori_loop(0, CH0 // 16, grp, 0)
            pltpu.sync_copy(v8c, v8.at[pl.ds(base, CH0)])

        return carry

    lax.fori_loop(0, nmax, body, 0)


def _stage1_body(nch, verts8, fx, fy, fz, fn, idx3, rows, fnc,
                 isem, gsem, ssem):
    wid = _wid()
    nmax = (nch + NW - 1) // NW
    if nmax % 2:
        nmax += 1
    iota = lax.iota(jnp.int32, 16)
    fsrcs = [fx, fy, fz]

    def valid(i):
        return (wid + i * NW) < nch

    def load_idx(i, s):
        @pl.when(valid(i))
        def _():
            off = (wid + i * NW) * CH1
            for j in range(3):
                pltpu.async_copy(fsrcs[j].at[pl.ds(off, CH1)], idx3.at[s, j],
                                 isem.at[s])

    def start_gather(i, s):
        @pl.when(valid(i))
        def _():
            for j in range(3):
                pltpu.make_async_copy(fsrcs[j].at[pl.ds(0, CH1)],
                                      idx3.at[s, j], isem.at[s]).wait()
            for j in range(3):
                pltpu.async_copy(verts8.at[idx3.at[s, j]], rows.at[s, j],
                                 gsem.at[s])

    def consume(i, s):
        @pl.when(valid(i))
        def _():
            off = (wid + i * NW) * CH1
            for j in range(3):
                pltpu.make_async_copy(verts8.at[idx3.at[s, j]],
                                      rows.at[s, j], gsem.at[s]).wait()

            def asm(g, c2):
                g16 = iota + 16 * g
                for j in range(3):
                    for cc in range(3):
                        v = plsc.load_gather(rows.at[s, j], [g16, jnp.full(
                            (16,), cc, jnp.int32)])
                        plsc.store_scatter(
                            fnc.at[s],
                            [g16, jnp.full((16,), 4 * j + cc, jnp.int32)], v)
                return c2

            lax.fori_loop(0, CH1 // 16, asm, 0)
            pltpu.async_copy(fnc.at[s], fn.at[pl.ds(off, CH1)], ssem.at[s])

    def drain(i, s):
        @pl.when(jnp.logical_and(i >= 0, valid(i)))
        def _():
            pltpu.make_async_copy(fnc.at[s], fn.at[pl.ds(0, CH1)],
                                  ssem.at[s]).wait()

    load_idx(0, 0)
    load_idx(1, 1)
    start_gather(0, 0)

    def slot(i, s, s1):
        start_gather(i + 1, s1)
        drain(i - 2, s)
        consume(i, s)
        load_idx(i + 2, s)

    def pair(i2, carry):
        i = 2 * i2
        slot(i, 0, 1)
        slot(i + 1, 1, 0)
        return carry

    lax.fori_loop(0, nmax // 2, pair, 0)
    drain(nmax - 2, 0)
    drain(nmax - 1, 1)


def _stage2_body(npix, hw, w, fn, pix, bary, out,
                 pidx, fnrows, bary_v, ov, gsem, psem, bsem, ssem):
    wid = _wid()
    per_w = npix // NW
    nchunk = per_w // CH2
    base = wid * per_w
    hw3 = 3 * hw
    gpr = w // 16

    iota = lax.iota(jnp.int32, 16)
    colc = [[jnp.full((16,), 4 * j + c, jnp.int32) for j in range(3)]
            for c in range(3)]

    def load_pix(i, s):
        off = base + i * CH2
        pltpu.async_copy(pix.at[pl.ds(off, CH2)], pidx.at[s], psem.at[s])

    def wait_pix(i, s):
        off = base + i * CH2
        pltpu.make_async_copy(pix.at[pl.ds(off, CH2)], pidx.at[s],
                              psem.at[s]).wait()

    def load_bary(i, s):
        off = base + i * CH2
        b = off // hw
        r = off - b * hw
        bsrc = b * hw3 + (r // w) * (3 * w)
        pltpu.async_copy(bary.at[pl.ds(bsrc, 3 * CH2)], bary_v.at[s],
                         bsem.at[s])

    def wait_bary(s):
        pltpu.make_async_copy(bary.at[pl.ds(0, 3 * CH2)], bary_v.at[s],
                              bsem.at[s]).wait()

    def issue_gather(s):
        pltpu.async_copy(fn.at[pidx.at[s]], fnrows.at[s], gsem.at[s])

    def wait_gather(s):
        pltpu.make_async_copy(fn.at[pidx.at[s]], fnrows.at[s],
                              gsem.at[s]).wait()

    def drain_ov(s):
        for c in range(3):
            pltpu.make_async_copy(ov.at[s, c], out.at[pl.ds(0, CH2)],
                                  ssem.at[s]).wait()

    def compute(i, s):
        off = base + i * CH2
        b = off // hw
        r = off - b * hw

        def grp(g, c2):
            lr = g // gpr
            wb = 16 * g - lr * w
            n_vec = jnp.full((16,), 16 * g, jnp.int32) + iota
            bb = lr * (3 * w) + wb
            wj = [bary_v.at[s][pl.ds(bb + j * w, 16)] for j in range(3)]
            for c in range(3):
                acc = None
                for j in range(3):
                    f = plsc.load_gather(fnrows.at[s], [n_vec, colc[c][j]])
                    acc = f * wj[j] if acc is None else acc + f * wj[j]
                ov.at[s, c][pl.ds(16 * g, 16)] = acc
            return c2

        lax.fori_loop(0, CH2 // 16, grp, 0)
        for c in range(3):
            pltpu.async_copy(ov.at[s, c],
                             out.at[pl.ds(b * hw3 + c * hw + r, CH2)],
                             ssem.at[s])

    def slot(i, s, s1):
        @pl.when(i + 1 < nchunk)
        def _():
            wait_pix(i + 1, s1)
            issue_gather(s1)
        wait_gather(s)
        wait_bary(s)

        @pl.when(i >= 2)
        def _():
            drain_ov(s)
        compute(i, s)

        @pl.when(i + 2 < nchunk)
        def _():
            load_pix(i + 2, s)
            load_bary(i + 2, s)

    load_pix(0, 0)
    load_bary(0, 0)
    load_pix(1, 1)
    load_bary(1, 1)
    wait_pix(0, 0)
    issue_gather(0)

    def pair(i2, carry):
        i = 2 * i2
        slot(i, 0, 1)
        slot(i + 1, 1, 0)
        return carry

    lax.fori_loop(0, nchunk // 2, pair, 0)
    drain_ov(0)
    drain_ov(1)


def kernel(verts_normals, faces, pix_to_face, bary_coords):
    B, V, _ = verts_normals.shape
    F = faces.shape[0]
    _, H, W, K = pix_to_face.shape
    P = B * H * W

    vc = verts_normals.transpose(2, 0, 1).reshape(3, B * V)
    faces32 = faces.astype(jnp.int32)
    fx, fy, fz = faces32[:, 0], faces32[:, 1], faces32[:, 2]
    pix = pix_to_face[..., 0].astype(jnp.int32).reshape(P)
    bary = bary_coords.transpose(0, 1, 4, 3, 2).reshape(B * H * 3 * W)

    mesh = plsc.VectorSubcoreMesh(core_axis_name="c", subcore_axis_name="s",
                                  num_cores=2, num_subcores=16)
    params = pltpu.CompilerParams(use_tc_tiling_on_sc=False,
                                  needs_layout_passes=False)
    nch = F // CH1

    nv = B * V
    stage0 = pl.kernel(
        functools.partial(_stage0_body, nv),
        out_type=jax.ShapeDtypeStruct((nv, 8), jnp.float32),
        mesh=mesh,
        compiler_params=params,
        scratch_types=[
            pltpu.VMEM((CH0,), jnp.float32),
            pltpu.VMEM((CH0,), jnp.float32),
            pltpu.VMEM((CH0,), jnp.float32),
            pltpu.VMEM((CH0, 8), jnp.float32),
        ],
    )
    verts8 = stage0(vc)

    stage1 = pl.kernel(
        functools.partial(_stage1_body, nch),
        out_type=jax.ShapeDtypeStruct((F, 16), jnp.float32),
        mesh=mesh,
        compiler_params=params,
        scratch_types=[
            pltpu.VMEM((2, 3, CH1), jnp.int32),
            pltpu.VMEM((2, 3, CH1, 8), jnp.float32),
            pltpu.VMEM((2, CH1, 16), jnp.float32),
            pltpu.SemaphoreType.DMA((2,)),
            pltpu.SemaphoreType.DMA((2,)),
            pltpu.SemaphoreType.DMA((2,)),
        ],
    )
    fn = stage1(verts8, fx, fy, fz)

    stage2 = pl.kernel(
        functools.partial(_stage2_body, P, H * W, W),
        out_type=jax.ShapeDtypeStruct((P * 3,), jnp.float32),
        mesh=mesh,
        compiler_params=params,
        scratch_types=[
            pltpu.VMEM((2, CH2), jnp.int32),
            pltpu.VMEM((2, CH2, 16), jnp.float32),
            pltpu.VMEM((2, CH2 * 3), jnp.float32),
            pltpu.VMEM((2, 3, CH2), jnp.float32),
            pltpu.SemaphoreType.DMA((2,)),
            pltpu.SemaphoreType.DMA((2,)),
            pltpu.SemaphoreType.DMA((2,)),
            pltpu.SemaphoreType.DMA((2,)),
        ],
    )
    out = stage2(fn, pix, bary)
    return out.reshape(B, 3, H, W).transpose(0, 2, 3, 1)
